# baseline (speedup 1.0000x reference)
import functools

import jax
import jax.numpy as jnp
from jax import lax
from jax.experimental import pallas as pl
from jax.experimental.pallas import tpu as pltpu
from jax.experimental.pallas import tpu_sc as plsc

NUM_EXPERTS = 16
MAX_E = 4
MIN_E = 1
TB = 512
NC, NS = 2, 16
NW = NC * NS
T_TOTAL = 8192
RPW = (T_TOTAL // 8) // NW

_NT = (((1,), (1,)), ((), ()))


def _moe_tc_body(x_ref, w1_ref, wr_ref, b1_ref, br_ref, w2_ref, b2_ref,
                 logits_ref, probs_ref, keep_ref, conf_ref):
    x = x_ref[...]
    acc1 = lax.dot_general(x, w1_ref[...], _NT,
                           preferred_element_type=jnp.float32)
    h = jnp.maximum(acc1 + b1_ref[...], 0.0)
    logits = lax.dot_general(x, wr_ref[...], _NT,
                             preferred_element_type=jnp.float32)[:, :NUM_EXPERTS]
    logits = logits + br_ref[...]
    logits_ref[...] = logits

    m = jnp.max(logits, axis=1, keepdims=True)
    e = jnp.exp(logits - m)
    probs_ref[...] = e / jnp.sum(e, axis=1, keepdims=True)

    conf_pre = lax.dot_general(h, w2_ref[...], _NT,
                               preferred_element_type=jnp.float32)[:, 0:1]
    conf = jax.nn.sigmoid(conf_pre + b2_ref[...])
    conf_ref[...] = conf

    dyn = jnp.clip(
        jnp.round(MIN_E + (MAX_E - MIN_E) * (1.0 - conf)).astype(jnp.int32),
        MIN_E, MAX_E)
    lane = lax.broadcasted_iota(jnp.int32, (TB, NUM_EXPERTS), 1)
    keep_ref[...] = (lane < dyn).astype(jnp.float32)


_GDN = lax.GatherDimensionNumbers(
    offset_dims=(), collapsed_slice_dims=(0,), start_index_map=(0,))


def _gath(x, idx):
    return lax.gather(x, idx[:, None], _GDN, (1,),
                      mode=lax.GatherScatterMode.PROMISE_IN_BOUNDS)


def _sc_sort_body(ppk_hbm, mpk_hbm, wpk_hbm, ipk_hbm, pv, mv, wv, iv):
    wid = lax.axis_index("s") * NC + lax.axis_index("c")
    row0 = wid * RPW
    pltpu.sync_copy(ppk_hbm.at[pl.ds(row0, RPW), :], pv)
    pltpu.sync_copy(mpk_hbm.at[pl.ds(row0, RPW), :], mv)
    ii = lax.iota(jnp.int32, 16)
    rots = {s: (ii + s) % 16 for s in (8, 4, 2, 1)}
    lane_eq = [ii == r for r in range(MAX_E)]
    zi = ii - ii

    def _all_max(x):
        for s in (8, 4, 2, 1):
            x = jnp.maximum(x, _gath(x, rots[s]))
        return x

    def _all_min(x):
        for s in (8, 4, 2, 1):
            x = jnp.minimum(x, _gath(x, rots[s]))
        return x

    def row_body(r, carry):
        for j in range(8):
            p = pv[r, pl.ds(16 * j, 16)]
            m = mv[r, pl.ds(16 * j, 16)]
            ow = p * 0.0
            oi = zi
            for r_out in range(MAX_E):
                mx = _all_max(p)
                am = _all_min(jnp.where(p == mx, ii, NUM_EXPERTS))
                ow = jnp.where(lane_eq[r_out], mx, ow)
                oi = jnp.where(lane_eq[r_out], am, oi)
                p = jnp.where(ii == am, -1e30, p)
            wv[r, pl.ds(16 * j, 16)] = ow * m
            iv[r, pl.ds(16 * j, 16)] = jnp.where(m > 0.5, oi, 0)
        return carry

    lax.fori_loop(0, RPW, row_body, 0)
    pltpu.sync_copy(wv, wpk_hbm.at[pl.ds(row0, RPW), :])
    pltpu.sync_copy(iv, ipk_hbm.at[pl.ds(row0, RPW), :])


@functools.partial(jax.jit, static_argnames=())
def kernel(hidden_states, Wr, br, W1, b1, W2, b2):
    B, S, H = hidden_states.shape
    T = B * S
    flat = hidden_states.reshape(T, H)

    wr_pad = jnp.pad(Wr, ((0, 128 - NUM_EXPERTS), (0, 0)))
    w2_pad = jnp.pad(W2, ((0, 127), (0, 0)))

    grid = (T // TB,)
    logits, probs, keep, conf = pl.pallas_call(
        _moe_tc_body,
        grid=grid,
        in_specs=[
            pl.BlockSpec((TB, H), lambda i: (i, 0)),
            pl.BlockSpec((1024, H), lambda i: (0, 0)),
            pl.BlockSpec((128, H), lambda i: (0, 0)),
            pl.BlockSpec((1, 1024), lambda i: (0, 0)),
            pl.BlockSpec((1, NUM_EXPERTS), lambda i: (0, 0)),
            pl.BlockSpec((128, 1024), lambda i: (0, 0)),
            pl.BlockSpec((1, 1), lambda i: (0, 0)),
        ],
        out_specs=[
            pl.BlockSpec((TB, NUM_EXPERTS), lambda i: (i, 0)),
            pl.BlockSpec((TB, NUM_EXPERTS), lambda i: (i, 0)),
            pl.BlockSpec((TB, NUM_EXPERTS), lambda i: (i, 0)),
            pl.BlockSpec((TB, 1), lambda i: (i, 0)),
        ],
        out_shape=[
            jax.ShapeDtypeStruct((T, NUM_EXPERTS), jnp.float32),
            jax.ShapeDtypeStruct((T, NUM_EXPERTS), jnp.float32),
            jax.ShapeDtypeStruct((T, NUM_EXPERTS), jnp.float32),
            jax.ShapeDtypeStruct((T, 1), jnp.float32),
        ],
    )(flat, W1, wr_pad, b1.reshape(1, 1024), br.reshape(1, NUM_EXPERTS),
      w2_pad, b2.reshape(1, 1))
    ppk = probs.reshape(T // 8, 128)
    mpk = keep.reshape(T // 8, 128)

    sc_sort = pl.kernel(
        _sc_sort_body,
        out_type=[
            jax.ShapeDtypeStruct((T // 8, 128), jnp.float32),
            jax.ShapeDtypeStruct((T // 8, 128), jnp.int32),
        ],
        mesh=plsc.VectorSubcoreMesh(core_axis_name="c", subcore_axis_name="s"),
        scratch_types=[
            pltpu.VMEM((RPW, 128), jnp.float32),
            pltpu.VMEM((RPW, 128), jnp.float32),
            pltpu.VMEM((RPW, 128), jnp.float32),
            pltpu.VMEM((RPW, 128), jnp.int32),
        ],
    )
    wpk, ipk = sc_sort(ppk, mpk)

    selected_weights = wpk.reshape(T, NUM_EXPERTS)[:, :MAX_E].reshape(B, S, MAX_E)
    selected_indices = (ipk.reshape(T, NUM_EXPERTS)[:, :MAX_E]
                        .astype(jnp.int64).reshape(B, S, MAX_E))
    confidence = conf.reshape(T)
    return selected_weights, selected_indices, confidence, logits
